# Initial kernel scaffold; baseline (speedup 1.0000x reference)
import jax
import jax.numpy as jnp
from jax.experimental import pallas as pl


def kernel(logits, temperature, top_k):
    raise NotImplementedError("write your pallas kernel here")


# speedup vs baseline: 2.6304x; 2.6304x over previous
import functools

import jax
import jax.numpy as jnp
from jax import lax
from jax.experimental import pallas as pl
from jax.experimental.pallas import tpu as pltpu
from jax.experimental.pallas import tpu_sc as plsc

_B = 64
_V = 1_000_000
_NC = 2
_NS = 16
_L = 16
_NW = _NC * _NS
_ROWS_PER_W = _B // _NW
_CHUNK = 40_000
_NCHUNK = _V // _CHUNK
_UNROLL = 4
_VECS = _CHUNK // _L
_STEPS = _VECS // _UNROLL
_OUT_PAD = 16

_mesh = plsc.VectorSubcoreMesh(core_axis_name="c", subcore_axis_name="s")


@functools.partial(
    pl.kernel,
    mesh=_mesh,
    out_type=jax.ShapeDtypeStruct((_NW, _OUT_PAD), jnp.int32),
    scratch_types=[
        pltpu.VMEM((_CHUNK,), jnp.float32),
        pltpu.VMEM((_CHUNK,), jnp.float32),
        pltpu.VMEM((_OUT_PAD,), jnp.int32),
        pltpu.SemaphoreType.DMA,
        pltpu.SemaphoreType.DMA,
    ],
    compiler_params=pltpu.CompilerParams(
        use_tc_tiling_on_sc=False, needs_layout_passes=False
    ),
)
def _sc_row_argmax(x_hbm, out_hbm, buf0, buf1, stage, sem0, sem1):
    wid = lax.axis_index("c") * _NS + lax.axis_index("s")
    bufs = (buf0, buf1)
    sems = (sem0, sem1)

    bests = []
    for r in range(_ROWS_PER_W):
        row = wid * _ROWS_PER_W + r

        def _start(g, row=row):
            return pltpu.async_copy(
                x_hbm.at[row, pl.ds(g * _CHUNK, _CHUNK)], bufs[g % 2], sems[g % 2]
            )

        pending = _start(0)
        acc_v = tuple(jnp.full((_L,), -jnp.inf, jnp.float32) for _ in range(_UNROLL))
        acc_j = tuple(jnp.zeros((_L,), jnp.int32) for _ in range(_UNROLL))

        for g in range(_NCHUNK):
            pending.wait()
            if g + 1 < _NCHUNK:
                pending = _start(g + 1)
            buf = bufs[g % 2]
            base = g * _VECS

            def _body(i, carry, buf=buf, base=base):
                vs, js = carry
                vs, js = list(vs), list(js)
                for a in range(_UNROLL):
                    j = i * _UNROLL + a
                    v = buf[pl.ds(j * _L, _L)]
                    jvec = jnp.full((_L,), base + j, jnp.int32)
                    m = v > vs[a]
                    vs[a] = jnp.where(m, v, vs[a])
                    js[a] = jnp.where(m, jvec, js[a])
                return tuple(vs), tuple(js)

            acc_v, acc_j = lax.fori_loop(0, _STEPS, _body, (acc_v, acc_j))

        bv, bj = acc_v[0], acc_j[0]
        for a in range(1, _UNROLL):
            take = (acc_v[a] > bv) | ((acc_v[a] == bv) & (acc_j[a] < bj))
            bv = jnp.where(take, acc_v[a], bv)
            bj = jnp.where(take, acc_j[a], bj)

        idx = bj * _L + lax.iota(jnp.int32, _L)
        mx = jnp.max(bv)
        best = jnp.min(jnp.where(bv == mx, idx, jnp.int32(2**30)))
        bests.append(best)

    lane = lax.iota(jnp.int32, _L)
    res = jnp.zeros((_L,), jnp.int32)
    for r, b in enumerate(bests):
        res = jnp.where(lane == r, jnp.full((_L,), b, jnp.int32), res)
    stage[pl.ds(0, _L)] = res
    pltpu.sync_copy(stage, out_hbm.at[wid])


def kernel(logits, temperature, top_k):
    del temperature, top_k
    x = logits[:, -1, :]
    raw = _sc_row_argmax(x)
    return raw[:, :_ROWS_PER_W].reshape(_B, 1)
